# baseline (speedup 1.0000x reference)
import functools

import jax
import jax.numpy as jnp
from jax import lax
from jax.experimental import pallas as pl
from jax.experimental.pallas import tpu as pltpu

_MS = 2.0


def _lse_body(xt_ref, t_ref, out_ref, m_s, s_s, v_s, *, nb, bc, n_rows):
    j = pl.program_id(0)

    @pl.when(j == 0)
    def _init():
        m_s[...] = jnp.full_like(m_s[...], -jnp.inf)
        s_s[...] = jnp.zeros_like(s_s[...])
        v_s[...] = jnp.zeros_like(v_s[...])

    def step(xb, cls):
        bm = jnp.max(xb, axis=0, keepdims=True)
        m_old = m_s[...]
        m_new = jnp.maximum(m_old, bm)
        s_s[...] = s_s[...] * jnp.exp(m_old - m_new) + jnp.sum(
            jnp.exp(xb - m_new), axis=0, keepdims=True)
        m_s[...] = m_new
        v_s[...] += jnp.sum(jnp.where(cls == t_ref[...], xb, 0.0), axis=0,
                            keepdims=True)

    cls = j * bc + lax.broadcasted_iota(jnp.int32, xt_ref.shape, 0)
    step(xt_ref[...], cls)

    @pl.when(j == nb - 1)
    def _last():
        vm2 = v_s[...] - _MS
        m = m_s[...]
        lse = m + jnp.log(s_s[...] - jnp.exp(vm2 + _MS - m) + jnp.exp(vm2 - m))
        out_ref[...] = jnp.sum(lse - vm2, keepdims=True) / n_rows


def kernel(x, target):
    B, C = x.shape
    xt = x.T
    bc = 5000
    assert C % bc == 0
    nb = C // bc
    body = functools.partial(_lse_body, nb=nb, bc=bc, n_rows=B)
    out = pl.pallas_call(
        body,
        grid=(nb,),
        in_specs=[
            pl.BlockSpec((bc, B), lambda j: (j, 0)),
            pl.BlockSpec((1, B), lambda j: (0, 0)),
        ],
        out_specs=pl.BlockSpec((1, 1), lambda j: (0, 0)),
        out_shape=jax.ShapeDtypeStruct((1, 1), jnp.float32),
        scratch_shapes=[
            pltpu.VMEM((1, B), jnp.float32),
            pltpu.VMEM((1, B), jnp.float32),
            pltpu.VMEM((1, B), jnp.float32),
        ],
    )(xt, target.reshape(1, B))
    return out[0, 0]


# speedup vs baseline: 1.0834x; 1.0834x over previous
import functools

import jax
import jax.numpy as jnp
from jax import lax
from jax.experimental import pallas as pl
from jax.experimental.pallas import tpu as pltpu
from jax.experimental.pallas import tpu_sc as plsc

_MS = 2.0


def _lse_body(xt_ref, vm2_ref, out_ref, m_s, s_s, *, nb, n_rows):
    j = pl.program_id(0)

    @pl.when(j == 0)
    def _init():
        m_s[...] = jnp.full_like(m_s[...], -jnp.inf)
        s_s[...] = jnp.zeros_like(s_s[...])

    xb = xt_ref[...]
    bm = jnp.max(xb, axis=0, keepdims=True)
    m_old = m_s[...]
    m_new = jnp.maximum(m_old, bm)
    s_s[...] = s_s[...] * jnp.exp(m_old - m_new) + jnp.sum(
        jnp.exp(xb - m_new), axis=0, keepdims=True)
    m_s[...] = m_new

    @pl.when(j == nb - 1)
    def _last():
        vm2 = vm2_ref[...]
        m = m_s[...]
        lse = m + jnp.log(s_s[...] - jnp.exp(vm2 + _MS - m) + jnp.exp(vm2 - m))
        out_ref[...] = jnp.sum(lse - vm2, keepdims=True) / n_rows


def _tc_loss(xt, vm2):
    C, B = xt.shape
    bc = 4000
    assert C % bc == 0
    nb = C // bc
    body = functools.partial(_lse_body, nb=nb, n_rows=B)
    return pl.pallas_call(
        body,
        grid=(nb,),
        in_specs=[
            pl.BlockSpec((bc, B), lambda j: (j, 0)),
            pl.BlockSpec((1, B), lambda j: (0, 0)),
        ],
        out_specs=pl.BlockSpec((1, 1), lambda j: (0, 0)),
        out_shape=jax.ShapeDtypeStruct((1, 1), jnp.float32),
        scratch_shapes=[
            pltpu.VMEM((1, B), jnp.float32),
            pltpu.VMEM((1, B), jnp.float32),
        ],
    )(xt, vm2)


def _sc_gather_vm2(z1, target, B):
    info = plsc.get_sparse_core_info()
    nc = info.num_cores
    nw = nc * info.num_subcores
    bpw = B // nw
    nbt8 = (B // 128) * 8

    @functools.partial(
        pl.kernel,
        mesh=plsc.VectorSubcoreMesh(core_axis_name="c", subcore_axis_name="s"),
        out_type=jax.ShapeDtypeStruct((B,), jnp.float32),
        scratch_types=[
            pltpu.VMEM((bpw,), jnp.int32),
            pltpu.VMEM((bpw,), jnp.int32),
            pltpu.VMEM((bpw,), jnp.float32),
            pltpu.VMEM((bpw,), jnp.float32),
            pltpu.SemaphoreType.DMA,
        ],
    )
    def k(z1_hbm, t_hbm, out_hbm, t_v, idx_v, val_v, out_v, sem):
        wid = lax.axis_index("s") * nc + lax.axis_index("c")
        base = wid * bpw
        pltpu.sync_copy(t_hbm.at[pl.ds(base, bpw)], t_v)
        for kk in range(bpw // 16):
            sl = pl.ds(kk * 16, 16)
            i16 = base + kk * 16 + lax.iota(jnp.int32, 16)
            t16 = t_v[sl]
            q16 = (t16 >> 3) * nbt8 + (i16 >> 7) * 8 + (t16 & 7)
            idx_v[sl] = q16 * 128 + (i16 & 127)
        pltpu.async_copy(z1_hbm.at[idx_v], val_v, sem).wait()
        for kk in range(bpw // 16):
            sl = pl.ds(kk * 16, 16)
            out_v[sl] = val_v[sl] - _MS
        pltpu.sync_copy(out_v, out_hbm.at[pl.ds(base, bpw)])

    return k(z1, target)


def kernel(x, target):
    B, C = x.shape
    xt = x.T
    z1 = (xt.reshape(C // 8, 8, B // 128, 128)
          .transpose(0, 2, 1, 3)
          .reshape(C * B))
    vm2 = _sc_gather_vm2(z1, target, B)
    out = _tc_loss(xt, vm2.reshape(1, B))
    return out[0, 0]
